# baseline (speedup 1.0000x reference)
import jax
import jax.numpy as jnp
from jax import lax
from jax.experimental import pallas as pl
from jax.experimental.pallas import tpu as pltpu
from jax.experimental.pallas import tpu_sc as plsc

_VOCAB_SIZE = 1000
_TBL = 2048
_N = 4096 * 20
_NW = 32
_PER_W = _N // _NW
_L = 16
_FULL = _VOCAB_SIZE // _L


def _lookup_body(inp_hbm, vocab_hbm, out_hbm, inp_v, vocab_v, inv_v, out_v,
                 inp_sem, vocab_sem):
    wid = lax.axis_index("s") * 2 + lax.axis_index("c")
    base = wid * _PER_W
    inp_dma = pltpu.async_copy(inp_hbm.at[pl.ds(base, _PER_W)], inp_v, inp_sem)
    vocab_dma = pltpu.async_copy(vocab_hbm, vocab_v, vocab_sem)

    lane = lax.iota(jnp.int32, _L)
    oov = jnp.full((_L,), _VOCAB_SIZE, jnp.int32)

    def init_step(i, carry):
        inv_v[pl.ds(i * _L, _L)] = oov
        return carry

    lax.fori_loop(0, _TBL // _L, init_step, 0, unroll=2)
    vocab_dma.wait()

    def scatter_step(j, carry):
        keys = plsc.bitcast(vocab_v[pl.ds(j * _L, _L)], jnp.int32)
        plsc.store_scatter(inv_v, [keys], lane + j * _L)
        return carry

    lax.fori_loop(0, _FULL, scatter_step, 0, unroll=2)
    tail_keys = plsc.bitcast(vocab_v[pl.ds(_VOCAB_SIZE - _L, _L)], jnp.int32)
    plsc.store_scatter(inv_v, [tail_keys], lane + (_VOCAB_SIZE - _L))

    inp_dma.wait()

    def gather_step(i, carry):
        off = i * _L
        toks = plsc.bitcast(inp_v[pl.ds(off, _L)], jnp.int32)
        out_v[pl.ds(off, _L)] = plsc.bitcast(
            plsc.load_gather(inv_v, [toks]), jnp.uint32)
        return carry

    lax.fori_loop(0, _PER_W // _L, gather_step, 0, unroll=2)

    pltpu.sync_copy(out_v, out_hbm.at[pl.ds(base, _PER_W)])


@jax.jit
def _lookup(flat_inputs, vocab):
    with jax.enable_x64(False):
        mesh = plsc.VectorSubcoreMesh(core_axis_name="c", subcore_axis_name="s")
        run = pl.kernel(
            _lookup_body,
            out_type=jax.ShapeDtypeStruct((_N,), jnp.uint32),
            mesh=mesh,
            scratch_types=[
                pltpu.VMEM((_PER_W,), jnp.uint32),
                pltpu.VMEM((_VOCAB_SIZE,), jnp.uint32),
                pltpu.VMEM((_TBL,), jnp.int32),
                pltpu.VMEM((_PER_W,), jnp.uint32),
                pltpu.SemaphoreType.DMA,
                pltpu.SemaphoreType.DMA,
            ],
            compiler_params=pltpu.CompilerParams(
                needs_layout_passes=False, skip_device_barrier=True),
        )
        return run(flat_inputs, vocab)


def kernel(inputs, vocabulary):
    flat = inputs.astype(jnp.uint32).T.reshape(-1)
    vocab = vocabulary.astype(jnp.uint32)
    out = _lookup(flat, vocab)
    return out.reshape(inputs.shape[::-1]).T.astype(jnp.int64)


# speedup vs baseline: 1.0556x; 1.0556x over previous
import jax
import jax.numpy as jnp
from jax import lax
from jax.experimental import pallas as pl
from jax.experimental.pallas import tpu as pltpu
from jax.experimental.pallas import tpu_sc as plsc

_VOCAB_SIZE = 1000
_TBL = 2048
_N = 4096 * 20
_NW = 32
_PER_W = _N // _NW
_L = 16
_FULL = _VOCAB_SIZE // _L


def _lookup_body(inp_hbm, vocab_hbm, out_hbm, inp_v, vocab_v, inv_v, out_v,
                 inp_sem, vocab_sem):
    wid = lax.axis_index("s") * 2 + lax.axis_index("c")
    base = wid * _PER_W
    inp_dma = pltpu.async_copy(inp_hbm.at[pl.ds(base, _PER_W)], inp_v, inp_sem)
    vocab_dma = pltpu.async_copy(vocab_hbm, vocab_v, vocab_sem)

    lane = lax.iota(jnp.int32, _L)
    oov = jnp.full((_L,), _VOCAB_SIZE, jnp.int32)

    @plsc.parallel_loop(0, _TBL // _L, unroll=8)
    def init_step(i):
        inv_v[pl.ds(i * _L, _L)] = oov

    vocab_dma.wait()

    @plsc.parallel_loop(0, _FULL, unroll=8)
    def scatter_step(j):
        keys = plsc.bitcast(vocab_v[pl.ds(j * _L, _L)], jnp.int32)
        plsc.store_scatter(inv_v, [keys], lane + j * _L)
    tail_keys = plsc.bitcast(vocab_v[pl.ds(_VOCAB_SIZE - _L, _L)], jnp.int32)
    plsc.store_scatter(inv_v, [tail_keys], lane + (_VOCAB_SIZE - _L))

    inp_dma.wait()

    @plsc.parallel_loop(0, _PER_W // _L, unroll=8)
    def gather_step(i):
        off = i * _L
        toks = plsc.bitcast(inp_v[pl.ds(off, _L)], jnp.int32)
        out_v[pl.ds(off, _L)] = plsc.bitcast(
            plsc.load_gather(inv_v, [toks]), jnp.uint32)

    pltpu.sync_copy(out_v, out_hbm.at[pl.ds(base, _PER_W)])


@jax.jit
def _lookup(flat_inputs, vocab):
    with jax.enable_x64(False):
        mesh = plsc.VectorSubcoreMesh(core_axis_name="c", subcore_axis_name="s")
        run = pl.kernel(
            _lookup_body,
            out_type=jax.ShapeDtypeStruct((_N,), jnp.uint32),
            mesh=mesh,
            scratch_types=[
                pltpu.VMEM((_PER_W,), jnp.uint32),
                pltpu.VMEM((_VOCAB_SIZE,), jnp.uint32),
                pltpu.VMEM((_TBL,), jnp.int32),
                pltpu.VMEM((_PER_W,), jnp.uint32),
                pltpu.SemaphoreType.DMA,
                pltpu.SemaphoreType.DMA,
            ],
            compiler_params=pltpu.CompilerParams(needs_layout_passes=False),
        )
        return run(flat_inputs, vocab)


def kernel(inputs, vocabulary):
    flat = inputs.astype(jnp.uint32).T.reshape(-1)
    vocab = vocabulary.astype(jnp.uint32)
    out = _lookup(flat, vocab)
    return out.reshape(inputs.shape[::-1]).T.astype(jnp.int64)
